# baseline (speedup 1.0000x reference)
import functools

import jax
import jax.numpy as jnp
from jax.experimental import pallas as pl
from jax.experimental.pallas import tpu as pltpu

_ALPHA = 0.25
_OHEM_RATIO = 0.7
_W = 1024
_NBUF = 6


def _focal_columns(xq, t, *, n_classes):
    w = xq.shape[1]
    n_chunks = n_classes // 8
    ridx0 = jax.lax.broadcasted_iota(jnp.int32, (8, w), 0)
    t_b = jnp.broadcast_to(t[None, :], (8, w))

    acc_m = jnp.full((8, w), -jnp.inf, jnp.float32)
    for c in range(n_chunks):
        acc_m = jnp.maximum(acc_m, xq[c * 8:(c + 1) * 8, :])
    m = jnp.max(acc_m, axis=0)
    m_b = jnp.broadcast_to(m[None, :], (8, w))

    acc_s = jnp.zeros((8, w), jnp.float32)
    for c in range(n_chunks):
        acc_s = acc_s + jnp.exp(xq[c * 8:(c + 1) * 8, :] - m_b)
    s = jnp.sum(acc_s, axis=0)

    acc_t = jnp.zeros((8, w), jnp.float32)
    for c in range(n_chunks):
        xc = xq[c * 8:(c + 1) * 8, :]
        hit = (ridx0 + c * 8) == t_b
        acc_t = acc_t + jnp.where(hit, xc, 0.0)
    tl = jnp.sum(acc_t, axis=0)

    lse = m + jnp.log(s)
    ce = lse - tl
    pt = jnp.exp(-ce)
    one_m = 1.0 - pt
    return _ALPHA * one_m * one_m * ce


def _fused_body(tgt_ref, xt_ref, out_ref, bufs_ref, facc_ref, sems_ref,
                *, n_classes, n_samples, k):
    n_slabs = n_samples // _W

    def start(j):
        pltpu.make_async_copy(
            xt_ref.at[:, pl.ds(j * _W, _W)],
            bufs_ref.at[j % _NBUF],
            sems_ref.at[j % _NBUF],
        ).start()

    def wait(j):
        pltpu.make_async_copy(
            xt_ref.at[:, pl.ds(j * _W, _W)],
            bufs_ref.at[j % _NBUF],
            sems_ref.at[j % _NBUF],
        ).wait()

    for j in range(_NBUF):
        start(j)
    for j in range(n_slabs):
        wait(j)
        t = tgt_ref[0, j * _W:(j + 1) * _W]
        f = _focal_columns(bufs_ref[j % _NBUF], t, n_classes=n_classes)
        if j + _NBUF < n_slabs:
            start(j + _NBUF)
        facc_ref[j, :] = f

    fall = facc_ref[...]
    bits = jax.lax.bitcast_convert_type(fall, jnp.int32)

    def step(j, prefix):
        cand = prefix | (jnp.int32(1) << (jnp.int32(30) - j))
        cnt = jnp.sum((bits >= cand).astype(jnp.int32))
        return jnp.where(cnt >= k, cand, prefix)

    thr = jax.lax.fori_loop(0, 31, step, jnp.int32(0))
    gt = bits > thr
    cnt_gt = jnp.sum(gt.astype(jnp.int32))
    sum_gt = jnp.sum(jnp.where(gt, fall, 0.0))
    thr_f = jnp.max(jnp.where(bits == thr, fall, 0.0))
    res = (
        sum_gt + (jnp.int32(k) - cnt_gt).astype(jnp.float32) * thr_f
    ) / jnp.float32(k)
    out_ref[...] = res[None, None]


def kernel(inputs, targets):
    n, c = inputs.shape
    k = int(_OHEM_RATIO * n)
    xt = inputs.T
    tgt = targets.astype(jnp.int32).reshape(1, n)
    body = functools.partial(_fused_body, n_classes=c, n_samples=n, k=k)
    out = pl.pallas_call(
        body,
        in_specs=[
            pl.BlockSpec(memory_space=pltpu.VMEM),
            pl.BlockSpec(memory_space=pltpu.HBM),
        ],
        out_specs=pl.BlockSpec(memory_space=pltpu.VMEM),
        out_shape=jax.ShapeDtypeStruct((1, 1), jnp.float32),
        scratch_shapes=[
            pltpu.VMEM((_NBUF, c, _W), jnp.float32),
            pltpu.VMEM((n // _W, _W), jnp.float32),
            pltpu.SemaphoreType.DMA((_NBUF,)),
        ],
    )(tgt, xt)
    return out[0, 0]


# speedup vs baseline: 1.0125x; 1.0125x over previous
import functools

import jax
import jax.numpy as jnp
from jax.experimental import pallas as pl
from jax.experimental.pallas import tpu as pltpu

_ALPHA = 0.25
_OHEM_RATIO = 0.7
_W = 1024
_NBUF = 6


def _focal_columns(xq, t, *, n_classes):
    w = xq.shape[1]
    n_chunks = n_classes // 8
    ridx0 = jax.lax.broadcasted_iota(jnp.int32, (8, w), 0)
    t_b = jnp.broadcast_to(t[None, :], (8, w))

    acc_m = jnp.full((8, w), -jnp.inf, jnp.float32)
    for c in range(n_chunks):
        acc_m = jnp.maximum(acc_m, xq[c * 8:(c + 1) * 8, :])
    m = jnp.max(acc_m, axis=0)
    m_b = jnp.broadcast_to(m[None, :], (8, w))

    acc_s = jnp.zeros((8, w), jnp.float32)
    acc_t = jnp.zeros((8, w), jnp.float32)
    for c in range(n_chunks):
        xc = xq[c * 8:(c + 1) * 8, :]
        hit = (ridx0 + c * 8) == t_b
        acc_s = acc_s + jnp.exp(xc - m_b)
        acc_t = acc_t + jnp.where(hit, xc, 0.0)
    s = jnp.sum(acc_s, axis=0)
    tl = jnp.sum(acc_t, axis=0)

    lse = m + jnp.log(s)
    ce = lse - tl
    pt = jnp.exp(-ce)
    one_m = 1.0 - pt
    return _ALPHA * one_m * one_m * ce


def _fused_body(tgt_ref, xt_ref, out_ref, bufs_ref, facc_ref, sems_ref,
                *, n_classes, n_samples, k):
    n_slabs = n_samples // _W

    half = (n_classes // 16) * 8

    def _copies(j):
        b = j % _NBUF
        return (
            pltpu.make_async_copy(
                xt_ref.at[:half, pl.ds(j * _W, _W)],
                bufs_ref.at[b, :half],
                sems_ref.at[b, 0],
            ),
            pltpu.make_async_copy(
                xt_ref.at[half:, pl.ds(j * _W, _W)],
                bufs_ref.at[b, half:],
                sems_ref.at[b, 1],
            ),
        )

    def start(j):
        for cp in _copies(j):
            cp.start()

    def wait(j):
        for cp in _copies(j):
            cp.wait()

    for j in range(_NBUF):
        start(j)
    for j in range(n_slabs):
        wait(j)
        t = tgt_ref[0, j * _W:(j + 1) * _W]
        f = _focal_columns(bufs_ref[j % _NBUF], t, n_classes=n_classes)
        if j + _NBUF < n_slabs:
            start(j + _NBUF)
        facc_ref[j, :] = f

    fall = facc_ref[...]
    bits = jax.lax.bitcast_convert_type(fall, jnp.int32)

    def step(j, prefix):
        cand = prefix | (jnp.int32(1) << (jnp.int32(30) - j))
        cnt = jnp.sum((bits >= cand).astype(jnp.int32))
        return jnp.where(cnt >= k, cand, prefix)

    thr = jax.lax.fori_loop(0, 31, step, jnp.int32(0))
    gt = bits > thr
    cnt_gt = jnp.sum(gt.astype(jnp.int32))
    sum_gt = jnp.sum(jnp.where(gt, fall, 0.0))
    thr_f = jnp.max(jnp.where(bits == thr, fall, 0.0))
    res = (
        sum_gt + (jnp.int32(k) - cnt_gt).astype(jnp.float32) * thr_f
    ) / jnp.float32(k)
    out_ref[...] = res[None, None]


def kernel(inputs, targets):
    n, c = inputs.shape
    k = int(_OHEM_RATIO * n)
    xt = inputs.T
    tgt = targets.astype(jnp.int32).reshape(1, n)
    body = functools.partial(_fused_body, n_classes=c, n_samples=n, k=k)
    out = pl.pallas_call(
        body,
        in_specs=[
            pl.BlockSpec(memory_space=pltpu.VMEM),
            pl.BlockSpec(memory_space=pltpu.HBM),
        ],
        out_specs=pl.BlockSpec(memory_space=pltpu.VMEM),
        out_shape=jax.ShapeDtypeStruct((1, 1), jnp.float32),
        scratch_shapes=[
            pltpu.VMEM((_NBUF, c, _W), jnp.float32),
            pltpu.VMEM((n // _W, _W), jnp.float32),
            pltpu.SemaphoreType.DMA((_NBUF, 2)),
        ],
    )(tgt, xt)
    return out[0, 0]


# speedup vs baseline: 1.0191x; 1.0066x over previous
import functools

import jax
import jax.numpy as jnp
from jax.experimental import pallas as pl
from jax.experimental.pallas import tpu as pltpu

_ALPHA = 0.25
_OHEM_RATIO = 0.7
_W = 1024
_NBUF = 8


def _focal_columns(xq, t, *, n_classes):
    w = xq.shape[1]
    n_chunks = n_classes // 8
    ridx0 = jax.lax.broadcasted_iota(jnp.int32, (8, w), 0)
    t_b = jnp.broadcast_to(t[None, :], (8, w))

    acc_m = jnp.full((8, w), -jnp.inf, jnp.float32)
    for c in range(n_chunks):
        acc_m = jnp.maximum(acc_m, xq[c * 8:(c + 1) * 8, :])
    m = jnp.max(acc_m, axis=0)
    m_b = jnp.broadcast_to(m[None, :], (8, w))

    acc_s = jnp.zeros((8, w), jnp.float32)
    acc_t = jnp.zeros((8, w), jnp.float32)
    for c in range(n_chunks):
        xc = xq[c * 8:(c + 1) * 8, :]
        hit = (ridx0 + c * 8) == t_b
        acc_s = acc_s + jnp.exp(xc - m_b)
        acc_t = acc_t + jnp.where(hit, xc, 0.0)
    s = jnp.sum(acc_s, axis=0)
    tl = jnp.sum(acc_t, axis=0)

    lse = m + jnp.log(s)
    ce = lse - tl
    pt = jnp.exp(-ce)
    one_m = 1.0 - pt
    return _ALPHA * one_m * one_m * ce


def _fused_body(tgt_ref, xt_ref, out_ref, bufs_ref, facc_ref, sems_ref,
                *, n_classes, n_samples, k):
    n_slabs = n_samples // _W

    def start(j):
        pltpu.make_async_copy(
            xt_ref.at[:, pl.ds(j * _W, _W)],
            bufs_ref.at[j % _NBUF],
            sems_ref.at[j % _NBUF],
        ).start()

    def wait(j):
        pltpu.make_async_copy(
            xt_ref.at[:, pl.ds(j * _W, _W)],
            bufs_ref.at[j % _NBUF],
            sems_ref.at[j % _NBUF],
        ).wait()

    _lead = _NBUF - 2
    for j in range(_lead):
        start(j)
    for j in range(n_slabs):
        wait(j)
        if j + _lead < n_slabs:
            start(j + _lead)
        t = tgt_ref[0, j * _W:(j + 1) * _W]
        f = _focal_columns(bufs_ref[j % _NBUF], t, n_classes=n_classes)
        facc_ref[j, :] = f

    fall = facc_ref[...]
    bits = jax.lax.bitcast_convert_type(fall, jnp.int32)

    def step(j, prefix):
        cand = prefix | (jnp.int32(1) << (jnp.int32(30) - j))
        cnt = jnp.sum((bits >= cand).astype(jnp.int32))
        return jnp.where(cnt >= k, cand, prefix)

    thr = jax.lax.fori_loop(0, 31, step, jnp.int32(0))
    gt = bits > thr
    cnt_gt = jnp.sum(gt.astype(jnp.int32))
    sum_gt = jnp.sum(jnp.where(gt, fall, 0.0))
    thr_f = jnp.max(jnp.where(bits == thr, fall, 0.0))
    res = (
        sum_gt + (jnp.int32(k) - cnt_gt).astype(jnp.float32) * thr_f
    ) / jnp.float32(k)
    out_ref[...] = res[None, None]


def kernel(inputs, targets):
    n, c = inputs.shape
    k = int(_OHEM_RATIO * n)
    xt = inputs.T
    tgt = targets.astype(jnp.int32).reshape(1, n)
    body = functools.partial(_fused_body, n_classes=c, n_samples=n, k=k)
    out = pl.pallas_call(
        body,
        in_specs=[
            pl.BlockSpec(memory_space=pltpu.VMEM),
            pl.BlockSpec(memory_space=pltpu.HBM),
        ],
        out_specs=pl.BlockSpec(memory_space=pltpu.VMEM),
        out_shape=jax.ShapeDtypeStruct((1, 1), jnp.float32),
        scratch_shapes=[
            pltpu.VMEM((_NBUF, c, _W), jnp.float32),
            pltpu.VMEM((n // _W, _W), jnp.float32),
            pltpu.SemaphoreType.DMA((_NBUF,)),
        ],
    )(tgt, xt)
    return out[0, 0]
